# baseline (speedup 1.0000x reference)
import functools

import jax
import jax.numpy as jnp
from jax import lax
from jax.experimental import pallas as pl
from jax.experimental.pallas import tpu as pltpu
from jax.experimental.pallas import tpu_sc as plsc

_NC = 2
_NS = 16
_NW = _NC * _NS


def _tc_table_transform(table, w, b, gamma, beta, blk=8000):
    v, d = table.shape
    o = w.shape[1]
    assert v % blk == 0

    def body(t_ref, w_ref, b_ref, g_ref, be_ref, out_ref):
        h = jnp.dot(t_ref[...], w_ref[...],
                    preferred_element_type=jnp.float32) + b_ref[...]
        mu = jnp.mean(h, axis=-1, keepdims=True)
        hc = h - mu
        var = jnp.mean(hc * hc, axis=-1, keepdims=True)
        out_ref[...] = hc * lax.rsqrt(var + 1e-5) * g_ref[...] + be_ref[...]

    return pl.pallas_call(
        body,
        grid=(v // blk,),
        in_specs=[
            pl.BlockSpec((blk, d), lambda i: (i, 0)),
            pl.BlockSpec((d, o), lambda i: (0, 0)),
            pl.BlockSpec((1, o), lambda i: (0, 0)),
            pl.BlockSpec((1, o), lambda i: (0, 0)),
            pl.BlockSpec((1, o), lambda i: (0, 0)),
        ],
        out_specs=pl.BlockSpec((blk, o), lambda i: (i, 0)),
        out_shape=jax.ShapeDtypeStruct((v, o), jnp.float32),
    )(table, w, b.reshape(1, o), gamma.reshape(1, o), beta.reshape(1, o))


def _sc_gather(ftable, idx, chunk=1280):
    n = idx.shape[0]
    d = ftable.shape[1]
    bpw = n // _NW
    nchunks = bpw // chunk
    assert bpw % chunk == 0 and bpw % 8 == 0

    mesh = plsc.VectorSubcoreMesh(core_axis_name="c", subcore_axis_name="s")

    @functools.partial(
        pl.kernel,
        mesh=mesh,
        compiler_params=pltpu.CompilerParams(use_tc_tiling_on_sc=False),
        out_type=jax.ShapeDtypeStruct((n, d), jnp.float32),
        scratch_types=[
            pltpu.VMEM((chunk,), jnp.int32),
            pltpu.VMEM((chunk, d), jnp.float32),
            pltpu.SemaphoreType.DMA,
        ],
    )
    def k(ftable_hbm, idx_hbm, out_hbm, idx_v, rows_v, sem):
        wid = lax.axis_index("s") * _NC + lax.axis_index("c")
        base = wid * bpw
        for j in range(nchunks):
            off = base + j * chunk
            pltpu.sync_copy(idx_hbm.at[pl.ds(off, chunk)], idx_v)
            pltpu.async_copy(ftable_hbm.at[idx_v], rows_v, sem).wait()
            pltpu.sync_copy(rows_v, out_hbm.at[pl.ds(off, chunk)])

    return k(ftable, idx)


def kernel(texts, table, W, b, gamma, beta):
    bsz, t = texts.shape
    out_dim = W.shape[1]
    idx = texts.reshape(-1)
    ftable = _tc_table_transform(table, W, b, gamma, beta)
    out = _sc_gather(ftable, idx)
    return out.reshape(bsz, t, out_dim)


# speedup vs baseline: 2.3268x; 2.3268x over previous
import functools

import jax
import jax.numpy as jnp
from jax import lax
from jax.experimental import pallas as pl
from jax.experimental.pallas import tpu as pltpu
from jax.experimental.pallas import tpu_sc as plsc

_NC = 2
_NS = 16
_NW = _NC * _NS


def _sc_gather(table, idx, chunk=2560):
    n = idx.shape[0]
    d = table.shape[1]
    bpw = n // _NW
    nchunks = bpw // chunk
    assert bpw % chunk == 0 and bpw % 8 == 0

    mesh = plsc.VectorSubcoreMesh(core_axis_name="c", subcore_axis_name="s")

    @functools.partial(
        pl.kernel,
        mesh=mesh,
        compiler_params=pltpu.CompilerParams(use_tc_tiling_on_sc=False),
        out_type=jax.ShapeDtypeStruct((n, d), jnp.float32),
        scratch_types=[
            pltpu.VMEM((chunk,), jnp.int32),
            pltpu.VMEM((chunk, d), jnp.float32),
            pltpu.SemaphoreType.DMA,
        ],
    )
    def k(table_hbm, idx_hbm, out_hbm, idx_v, rows_v, sem):
        wid = lax.axis_index("s") * _NC + lax.axis_index("c")
        base = wid * bpw
        for j in range(nchunks):
            off = base + j * chunk
            pltpu.sync_copy(idx_hbm.at[pl.ds(off, chunk)], idx_v)
            pltpu.async_copy(table_hbm.at[idx_v], rows_v, sem).wait()
            pltpu.sync_copy(rows_v, out_hbm.at[pl.ds(off, chunk)])

    return k(table, idx)


def _tc_transform_t(emb4, wt, bt, gt, bet, t, bsz):
    q = bsz // 4
    o = wt.shape[0]

    def body(e_ref, wt_ref, b_ref, g_ref, be_ref, out_ref):
        x4 = e_ref[0]
        for k in range(4):
            xk = x4[:, 32 * k:32 * (k + 1)]
            hk = lax.dot_general(
                wt_ref[...], xk, (((1,), (1,)), ((), ())),
                preferred_element_type=jnp.float32) + b_ref[...]
            mu = jnp.mean(hk, axis=0, keepdims=True)
            hc = hk - mu
            var = jnp.mean(hc * hc, axis=0, keepdims=True)
            out_ref[0, :, q * k:q * (k + 1)] = (
                hc * lax.rsqrt(var + 1e-5) * g_ref[...] + be_ref[...])

    return pl.pallas_call(
        body,
        grid=(t,),
        in_specs=[
            pl.BlockSpec((1, q, 128), lambda i: (i, 0, 0)),
            pl.BlockSpec((o, 32), lambda i: (0, 0)),
            pl.BlockSpec((o, 1), lambda i: (0, 0)),
            pl.BlockSpec((o, 1), lambda i: (0, 0)),
            pl.BlockSpec((o, 1), lambda i: (0, 0)),
        ],
        out_specs=pl.BlockSpec((1, o, bsz), lambda i: (i, 0, 0)),
        out_shape=jax.ShapeDtypeStruct((t, o, bsz), jnp.float32),
    )(emb4, wt, bt, gt, bet)


def kernel(texts, table, W, b, gamma, beta):
    bsz, t = texts.shape
    o = W.shape[1]
    q = bsz // 4
    idx = jnp.transpose(texts).reshape(t, 4, q)
    idx = jnp.transpose(idx, (0, 2, 1)).reshape(-1)
    emb = _sc_gather(table, idx)
    emb4 = emb.reshape(t, q, 128)
    wt = jnp.transpose(W)
    out_t = _tc_transform_t(emb4, wt, b.reshape(o, 1), gamma.reshape(o, 1),
                            beta.reshape(o, 1), t, bsz)
    return jnp.transpose(out_t, (2, 0, 1))
